# baseline (speedup 1.0000x reference)
import jax
import jax.numpy as jnp
from jax import lax
from jax.experimental import pallas as pl
from jax.experimental.pallas import tpu as pltpu
from jax.experimental.pallas import tpu_sc as plsc

B = 16384
D = 32
NC = 2
NS = 16
NW = NC * NS
CHUNK = B // NW
QROWS = CHUNK // 128
NITEMS = 1_000_000


def _body(user_hbm, item_hbm, u_hbm, sig_hbm, v_hbm, ub_hbm, ib_hbm,
          out_hbm, uidx, iidx, urows, vrows, ud, vtd, ubv, ibv, sig, outv,
          sem):
  wid = lax.axis_index("s") * NC + lax.axis_index("c")
  r0 = wid * QROWS
  base = wid * CHUNK
  iota = lax.iota(jnp.int32, 16)

  pltpu.sync_copy(user_hbm.at[pl.ds(r0, QROWS)], uidx)
  pltpu.sync_copy(item_hbm.at[pl.ds(r0, QROWS)], iidx)
  pltpu.sync_copy(sig_hbm, sig)

  copies = []
  for q in range(QROWS):
    copies.append(pltpu.async_copy(ub_hbm.at[uidx.at[q]], ubv.at[q], sem))
    copies.append(pltpu.async_copy(ib_hbm.at[iidx.at[q]], ibv.at[q], sem))
    copies.append(
        pltpu.async_copy(u_hbm.at[uidx.at[q]],
                         urows.at[pl.ds(q * 128, 128)], sem))
    copies.append(
        pltpu.async_copy(v_hbm.at[iidx.at[q]],
                         vrows.at[pl.ds(q * 128, 128)], sem))
  for cp in copies:
    cp.wait()

  dvec = iota * CHUNK

  def transpose(j, _):
    for h in range(2):
      idx = dvec + (h * 16 * CHUNK + j)
      plsc.store_scatter(ud, [idx], urows[j, pl.ds(h * 16, 16)])
      plsc.store_scatter(vtd, [idx], vrows[j, pl.ds(h * 16, 16)])
    return 0

  lax.fori_loop(0, CHUNK, transpose, 0)

  def compute(jc, _):
    row = jc // 8
    col = (jc % 8) * 16
    sig_lo = sig[pl.ds(0, 16)]
    sig_hi = sig[pl.ds(16, 16)]
    acc = ubv[row, pl.ds(col, 16)] + ibv[row, pl.ds(col, 16)]
    for d in range(D):
      sig_d = sig_lo[d] if d < 16 else sig_hi[d - 16]
      vt_chunk = vtd[pl.ds(d * CHUNK + jc * 16, 16)]
      u_chunk = ud[pl.ds(d * CHUNK + jc * 16, 16)]
      acc = acc + (sig_d * vt_chunk) * u_chunk
    outv[pl.ds(jc * 16, 16)] = acc
    return 0

  lax.fori_loop(0, CHUNK // 16, compute, 0)

  pltpu.sync_copy(outv, out_hbm.at[pl.ds(base, CHUNK)])


NSLAB = 7813
SLAB_K = (NSLAB + NW - 1) // NW


def _tbody(ut_hbm, vt_hbm, ou_hbm, ov_hbm, slab_u, slab_v, obu, obv, sem):
  wid = lax.axis_index("s") * NC + lax.axis_index("c")
  iota = lax.iota(jnp.int32, 16)
  lvec = iota * D

  def do_slab(k, _):
    c = wid + k * NW

    @pl.when(c < NSLAB)
    def _():
      c128 = pl.multiple_of(c * 128, 128)
      cp1 = pltpu.async_copy(ut_hbm.at[:, pl.ds(c128, 128)], slab_u, sem)
      cp2 = pltpu.async_copy(vt_hbm.at[:, pl.ds(c128, 128)], slab_v, sem)
      cp1.wait()
      cp2.wait()

      def trans_d(d, _):
        dsplat = jnp.full((16,), 0, jnp.int32) + d
        for lc in range(8):
          rows = lc * 16 + iota
          plsc.store_scatter(obu, [rows, dsplat],
                             slab_u[d, pl.ds(lc * 16, 16)])
          plsc.store_scatter(obv, [rows, dsplat],
                             slab_v[d, pl.ds(lc * 16, 16)])
        return 0

      lax.fori_loop(0, D, trans_d, 0)

      @pl.when(c < NSLAB - 1)
      def _():
        pltpu.sync_copy(obu, ou_hbm.at[pl.ds(c128, 128)])
        pltpu.sync_copy(obv, ov_hbm.at[pl.ds(c128, 128)])

      @pl.when(c == NSLAB - 1)
      def _():
        pltpu.sync_copy(obu.at[pl.ds(0, 64)], ou_hbm.at[pl.ds(c128, 64)])
        pltpu.sync_copy(obv.at[pl.ds(0, 64)], ov_hbm.at[pl.ds(c128, 64)])

    return 0

  lax.fori_loop(0, SLAB_K, do_slab, 0)


@jax.jit
def _sc_transpose(ut, vt):
  mesh = plsc.VectorSubcoreMesh(core_axis_name="c", subcore_axis_name="s",
                                num_cores=NC, num_subcores=NS)
  return pl.kernel(
      _tbody,
      out_type=[jax.ShapeDtypeStruct((NITEMS, D), jnp.float32),
                jax.ShapeDtypeStruct((NITEMS, D), jnp.float32)],
      mesh=mesh,
      compiler_params=pltpu.CompilerParams(needs_layout_passes=False,
                                           use_tc_tiling_on_sc=True),
      scratch_types=[
          pltpu.VMEM((D, 128), jnp.float32),
          pltpu.VMEM((D, 128), jnp.float32),
          pltpu.VMEM((128, D), jnp.float32),
          pltpu.VMEM((128, D), jnp.float32),
          pltpu.SemaphoreType.DMA,
      ],
  )(ut, vt)


@jax.jit
def _svd_predict(user2d, item2d, U, Sigma, V, user_bias, item_bias):
  mesh = plsc.VectorSubcoreMesh(core_axis_name="c", subcore_axis_name="s",
                                num_cores=NC, num_subcores=NS)
  return pl.kernel(
      _body,
      out_type=jax.ShapeDtypeStruct((B,), jnp.float32),
      mesh=mesh,
      compiler_params=pltpu.CompilerParams(needs_layout_passes=False,
                                           use_tc_tiling_on_sc=False),
      scratch_types=[
          pltpu.VMEM((QROWS, 128), jnp.int32),
          pltpu.VMEM((QROWS, 128), jnp.int32),
          pltpu.VMEM((CHUNK, D), jnp.float32),
          pltpu.VMEM((CHUNK, D), jnp.float32),
          pltpu.VMEM((CHUNK * D,), jnp.float32),
          pltpu.VMEM((CHUNK * D,), jnp.float32),
          pltpu.VMEM((QROWS, 128), jnp.float32),
          pltpu.VMEM((QROWS, 128), jnp.float32),
          pltpu.VMEM((D,), jnp.float32),
          pltpu.VMEM((CHUNK,), jnp.float32),
          pltpu.SemaphoreType.DMA,
      ],
  )(user2d, item2d, U, Sigma, V, user_bias, item_bias)


def kernel(user, item, U, Sigma, VT, user_bias, item_bias):
  user2d = user.reshape(B // 128, 128)
  item2d = item.reshape(B // 128, 128)
  u_bm, v_bm = _sc_transpose(U.T, VT)
  return _svd_predict(user2d, item2d, u_bm, Sigma, v_bm, user_bias,
                      item_bias)


# speedup vs baseline: 1.1327x; 1.1327x over previous
import jax
import jax.numpy as jnp
from jax import lax
from jax.experimental import pallas as pl
from jax.experimental.pallas import tpu as pltpu
from jax.experimental.pallas import tpu_sc as plsc

B = 16384
D = 32
NC = 2
NS = 16
NW = NC * NS
CHUNK = B // NW
QROWS = CHUNK // 128
NITEMS = 1_000_000


def _body(user_hbm, item_hbm, u_hbm, sig_hbm, v_hbm, ub_hbm, ib_hbm,
          out_hbm, uidx, iidx, urows, vrows, ud, vtd, ubv, ibv, sig, outv,
          sem):
  wid = lax.axis_index("s") * NC + lax.axis_index("c")
  r0 = wid * QROWS
  base = wid * CHUNK
  iota = lax.iota(jnp.int32, 16)

  pltpu.sync_copy(user_hbm.at[pl.ds(r0, QROWS)], uidx)
  pltpu.sync_copy(item_hbm.at[pl.ds(r0, QROWS)], iidx)
  pltpu.sync_copy(sig_hbm, sig)

  copies = []
  for q in range(QROWS):
    copies.append(pltpu.async_copy(ub_hbm.at[uidx.at[q]], ubv.at[q], sem))
    copies.append(pltpu.async_copy(ib_hbm.at[iidx.at[q]], ibv.at[q], sem))
    copies.append(
        pltpu.async_copy(u_hbm.at[uidx.at[q]],
                         urows.at[pl.ds(q * 128, 128)], sem))
    copies.append(
        pltpu.async_copy(v_hbm.at[iidx.at[q]],
                         vrows.at[pl.ds(q * 128, 128)], sem))
  for cp in copies:
    cp.wait()

  dvec = iota * CHUNK

  def transpose(j, _):
    for h in range(2):
      idx = dvec + (h * 16 * CHUNK + j)
      plsc.store_scatter(ud, [idx], urows[j, pl.ds(h * 16, 16)])
      plsc.store_scatter(vtd, [idx], vrows[j, pl.ds(h * 16, 16)])
    return 0

  lax.fori_loop(0, CHUNK, transpose, 0)

  def compute(jc, _):
    row = jc // 8
    col = (jc % 8) * 16
    sig_lo = sig[pl.ds(0, 16)]
    sig_hi = sig[pl.ds(16, 16)]
    acc = ubv[row, pl.ds(col, 16)] + ibv[row, pl.ds(col, 16)]
    for d in range(D):
      sig_d = sig_lo[d] if d < 16 else sig_hi[d - 16]
      vt_chunk = vtd[pl.ds(d * CHUNK + jc * 16, 16)]
      u_chunk = ud[pl.ds(d * CHUNK + jc * 16, 16)]
      acc = acc + (sig_d * vt_chunk) * u_chunk
    outv[pl.ds(jc * 16, 16)] = acc
    return 0

  lax.fori_loop(0, CHUNK // 16, compute, 0)

  pltpu.sync_copy(outv, out_hbm.at[pl.ds(base, CHUNK)])


GW = 256
NGRP = 3907
LASTC = 1000064 - GW
LASTW = NITEMS - LASTC
KMAX = (NGRP + NW - 1) // NW


def _tbody(ut_hbm, vt_hbm, ou_hbm, ov_hbm, su0, sv0, su1, sv1, obu, obv,
           sem0, sem1):
  wid = lax.axis_index("s") * NC + lax.axis_index("c")
  iota = lax.iota(jnp.int32, 16)
  bufs = ((su0, sv0, sem0), (su1, sv1, sem1))

  def col0(g):
    return pl.multiple_of(jnp.where(g == NGRP - 1, LASTC, g * GW), 128)

  def issue(g, b):
    su, sv, sem = bufs[b]

    @pl.when(g < NGRP)
    def _():
      c0 = col0(g)
      pltpu.async_copy(ut_hbm.at[:, pl.ds(c0, GW)], su, sem)
      pltpu.async_copy(vt_hbm.at[:, pl.ds(c0, GW)], sv, sem)

  def process(g, b):
    su, sv, sem = bufs[b]

    @pl.when(g < NGRP)
    def _():
      pltpu.make_async_copy(ut_hbm.at[:, pl.ds(0, GW)], su, sem).wait()
      pltpu.make_async_copy(vt_hbm.at[:, pl.ds(0, GW)], sv, sem).wait()

      def trans_d(d, _):
        dsplat = jnp.full((16,), 0, jnp.int32) + d
        for lc in range(GW // 16):
          rows = lc * 16 + iota
          plsc.store_scatter(obu, [rows, dsplat], su[d, pl.ds(lc * 16, 16)])
          plsc.store_scatter(obv, [rows, dsplat], sv[d, pl.ds(lc * 16, 16)])
        return 0

      lax.fori_loop(0, D, trans_d, 0)

      c0 = col0(g)

      @pl.when(g < NGRP - 1)
      def _():
        pltpu.sync_copy(obu, ou_hbm.at[pl.ds(c0, GW)])
        pltpu.sync_copy(obv, ov_hbm.at[pl.ds(c0, GW)])

      @pl.when(g == NGRP - 1)
      def _():
        pltpu.sync_copy(obu.at[pl.ds(0, LASTW)],
                        ou_hbm.at[pl.ds(c0, LASTW)])
        pltpu.sync_copy(obv.at[pl.ds(0, LASTW)],
                        ov_hbm.at[pl.ds(c0, LASTW)])

  issue(wid, 0)

  def step(kk, _):
    for b in range(2):
      k = kk * 2 + b
      g = wid + k * NW
      issue(g + NW, 1 - b)
      process(g, b)
    return 0

  lax.fori_loop(0, (KMAX + 1) // 2, step, 0)


@jax.jit
def _sc_transpose(ut, vt):
  mesh = plsc.VectorSubcoreMesh(core_axis_name="c", subcore_axis_name="s",
                                num_cores=NC, num_subcores=NS)
  return pl.kernel(
      _tbody,
      out_type=[jax.ShapeDtypeStruct((NITEMS, D), jnp.float32),
                jax.ShapeDtypeStruct((NITEMS, D), jnp.float32)],
      mesh=mesh,
      compiler_params=pltpu.CompilerParams(needs_layout_passes=False,
                                           use_tc_tiling_on_sc=True),
      scratch_types=[
          pltpu.VMEM((D, GW), jnp.float32),
          pltpu.VMEM((D, GW), jnp.float32),
          pltpu.VMEM((D, GW), jnp.float32),
          pltpu.VMEM((D, GW), jnp.float32),
          pltpu.VMEM((GW, D), jnp.float32),
          pltpu.VMEM((GW, D), jnp.float32),
          pltpu.SemaphoreType.DMA,
          pltpu.SemaphoreType.DMA,
      ],
  )(ut, vt)


@jax.jit
def _svd_predict(user2d, item2d, U, Sigma, V, user_bias, item_bias):
  mesh = plsc.VectorSubcoreMesh(core_axis_name="c", subcore_axis_name="s",
                                num_cores=NC, num_subcores=NS)
  return pl.kernel(
      _body,
      out_type=jax.ShapeDtypeStruct((B,), jnp.float32),
      mesh=mesh,
      compiler_params=pltpu.CompilerParams(needs_layout_passes=False,
                                           use_tc_tiling_on_sc=False),
      scratch_types=[
          pltpu.VMEM((QROWS, 128), jnp.int32),
          pltpu.VMEM((QROWS, 128), jnp.int32),
          pltpu.VMEM((CHUNK, D), jnp.float32),
          pltpu.VMEM((CHUNK, D), jnp.float32),
          pltpu.VMEM((CHUNK * D,), jnp.float32),
          pltpu.VMEM((CHUNK * D,), jnp.float32),
          pltpu.VMEM((QROWS, 128), jnp.float32),
          pltpu.VMEM((QROWS, 128), jnp.float32),
          pltpu.VMEM((D,), jnp.float32),
          pltpu.VMEM((CHUNK,), jnp.float32),
          pltpu.SemaphoreType.DMA,
      ],
  )(user2d, item2d, U, Sigma, V, user_bias, item_bias)


def kernel(user, item, U, Sigma, VT, user_bias, item_bias):
  user2d = user.reshape(B // 128, 128)
  item2d = item.reshape(B // 128, 128)
  u_bm, v_bm = _sc_transpose(U.T, VT)
  return _svd_predict(user2d, item2d, u_bm, Sigma, v_bm, user_bias,
                      item_bias)


# speedup vs baseline: 2.7775x; 2.4522x over previous
import jax
import jax.numpy as jnp
from jax import lax
from jax.experimental import pallas as pl
from jax.experimental.pallas import tpu as pltpu
from jax.experimental.pallas import tpu_sc as plsc

B = 16384
D = 32
NC = 2
NS = 16
NW = NC * NS
CHUNK = B // NW
QROWS = CHUNK // 128
NITEMS = 1_000_000


def _body(user_hbm, item_hbm, u_hbm, sig_hbm, v_hbm, ub_hbm, ib_hbm,
          out_hbm, uidx, iidx, urows, vrows, ud, vtd, ubv, ibv, sig, outv,
          sem):
  wid = lax.axis_index("s") * NC + lax.axis_index("c")
  r0 = wid * QROWS
  base = wid * CHUNK
  iota = lax.iota(jnp.int32, 16)

  pltpu.sync_copy(user_hbm.at[pl.ds(r0, QROWS)], uidx)
  pltpu.sync_copy(item_hbm.at[pl.ds(r0, QROWS)], iidx)
  pltpu.sync_copy(sig_hbm, sig)

  copies = []
  for q in range(QROWS):
    copies.append(pltpu.async_copy(ub_hbm.at[uidx.at[q]], ubv.at[q], sem))
    copies.append(pltpu.async_copy(ib_hbm.at[iidx.at[q]], ibv.at[q], sem))
    copies.append(
        pltpu.async_copy(u_hbm.at[uidx.at[q]],
                         urows.at[pl.ds(q * 128, 128)], sem))
    copies.append(
        pltpu.async_copy(v_hbm.at[iidx.at[q]],
                         vrows.at[pl.ds(q * 128, 128)], sem))
  for cp in copies:
    cp.wait()

  dvec = iota * CHUNK

  def transpose(j, _):
    for h in range(2):
      idx = dvec + (h * 16 * CHUNK + j)
      plsc.store_scatter(ud, [idx], urows[j, pl.ds(h * 16, 16)])
      plsc.store_scatter(vtd, [idx], vrows[j, pl.ds(h * 16, 16)])
    return 0

  lax.fori_loop(0, CHUNK, transpose, 0)

  def compute(jc, _):
    row = jc // 8
    col = (jc % 8) * 16
    sig_lo = sig[pl.ds(0, 16)]
    sig_hi = sig[pl.ds(16, 16)]
    acc = ubv[row, pl.ds(col, 16)] + ibv[row, pl.ds(col, 16)]
    for d in range(D):
      sig_d = sig_lo[d] if d < 16 else sig_hi[d - 16]
      vt_chunk = vtd[pl.ds(d * CHUNK + jc * 16, 16)]
      u_chunk = ud[pl.ds(d * CHUNK + jc * 16, 16)]
      acc = acc + (sig_d * vt_chunk) * u_chunk
    outv[pl.ds(jc * 16, 16)] = acc
    return 0

  lax.fori_loop(0, CHUNK // 16, compute, 0)

  pltpu.sync_copy(outv, out_hbm.at[pl.ds(base, CHUNK)])


@jax.jit
def _svd_predict(user2d, item2d, U, Sigma, V, user_bias, item_bias):
  mesh = plsc.VectorSubcoreMesh(core_axis_name="c", subcore_axis_name="s",
                                num_cores=NC, num_subcores=NS)
  return pl.kernel(
      _body,
      out_type=jax.ShapeDtypeStruct((B,), jnp.float32),
      mesh=mesh,
      compiler_params=pltpu.CompilerParams(needs_layout_passes=False,
                                           use_tc_tiling_on_sc=False),
      scratch_types=[
          pltpu.VMEM((QROWS, 128), jnp.int32),
          pltpu.VMEM((QROWS, 128), jnp.int32),
          pltpu.VMEM((CHUNK, D), jnp.float32),
          pltpu.VMEM((CHUNK, D), jnp.float32),
          pltpu.VMEM((CHUNK * D,), jnp.float32),
          pltpu.VMEM((CHUNK * D,), jnp.float32),
          pltpu.VMEM((QROWS, 128), jnp.float32),
          pltpu.VMEM((QROWS, 128), jnp.float32),
          pltpu.VMEM((D,), jnp.float32),
          pltpu.VMEM((CHUNK,), jnp.float32),
          pltpu.SemaphoreType.DMA,
      ],
  )(user2d, item2d, U, Sigma, V, user_bias, item_bias)


def kernel(user, item, U, Sigma, VT, user_bias, item_bias):
  user2d = user.reshape(B // 128, 128)
  item2d = item.reshape(B // 128, 128)
  return _svd_predict(user2d, item2d, U, Sigma, VT.T, user_bias, item_bias)
